# Initial kernel scaffold; baseline (speedup 1.0000x reference)
import jax
import jax.numpy as jnp
from jax import lax
from jax.experimental import pallas as pl
from jax.experimental.pallas import tpu as pltpu

E = 64
TOPK = 2
D = 1024
F = 1024
T = 64

_HI = lax.Precision.HIGHEST


def _router_kernel(hid_ref, rw_ref, gsched_ref, sched_ref, nact_ref):
    hid = hid_ref[...]
    rw = rw_ref[...]
    logits = lax.dot_general(hid, rw, (((1,), (1,)), ((), ())),
                             preferred_element_type=jnp.float32,
                             precision=_HI)
    iota_e = lax.broadcasted_iota(jnp.int32, (T, E), 1)
    m1 = jnp.max(logits, axis=1, keepdims=True)
    i1 = jnp.min(jnp.where(logits == m1, iota_e, E), axis=1, keepdims=True)
    s1 = iota_e == i1
    masked = jnp.where(s1, -jnp.inf, logits)
    m2 = jnp.max(masked, axis=1, keepdims=True)
    i2 = jnp.min(jnp.where(masked == m2, iota_e, E), axis=1, keepdims=True)
    s2 = iota_e == i2
    e2 = jnp.exp(m2 - m1)
    w1 = 1.0 / (1.0 + e2)
    w2 = e2 / (1.0 + e2)
    gates = jnp.where(s1, w1, 0.0) + jnp.where(s2, w2, 0.0)

    sel = (s1 | s2).astype(jnp.float32)
    ones_t = jnp.ones((T, 1), jnp.float32)
    cnt_col = lax.dot_general(sel, ones_t, (((0,), (0,)), ((), ())),
                              precision=_HI)
    active_col = cnt_col > 0.5
    af = active_col.astype(jnp.float32)
    ii = lax.broadcasted_iota(jnp.int32, (E, E), 0)
    jj = lax.broadcasted_iota(jnp.int32, (E, E), 1)
    lt = (ii <= jj).astype(jnp.float32)
    incl_col = lax.dot_general(lt, af, (((0,), (0,)), ((), ())),
                               precision=_HI)
    excl_col = (incl_col - af).astype(jnp.int32)
    slot = lax.broadcasted_iota(jnp.int32, (E, E), 1)
    m_mat = ((excl_col == slot) & active_col).astype(jnp.float32)
    iota_col = lax.broadcasted_iota(jnp.float32, (E, 1), 0)
    sched_row = lax.dot_general(iota_col, m_mat, (((0,), (0,)), ((), ())),
                                precision=_HI)
    sched_ref[...] = sched_row.astype(jnp.int32)
    nact_ref[...] = incl_col[E - 1:E, :].astype(jnp.int32)
    gsched_ref[...] = lax.dot_general(m_mat, gates, (((0,), (0,)), ((), ())),
                                      precision=_HI)


def _moe_kernel(sched_sref, nact_sref, hid_ref, w13_ref, w2_ref, gsched_ref,
                out_ref):
    i = pl.program_id(0)

    @pl.when(i == 0)
    def _init():
        out_ref[...] = jnp.zeros_like(out_ref)

    @pl.when(i < nact_sref[0])
    def _compute():
        hid = hid_ref[...]
        h = lax.dot_general(hid, w13_ref[0], (((1,), (1,)), ((), ())),
                            preferred_element_type=jnp.float32)
        gatep = h[:, :F]
        up = h[:, F:]
        act = gatep * (1.0 / (1.0 + jnp.exp(-gatep))) * up
        y = lax.dot_general(act, w2_ref[0], (((1,), (1,)), ((), ())),
                            preferred_element_type=jnp.float32)
        g = gsched_ref[0]
        out_ref[...] += y * g


def kernel(hidden_states, router_weight, w13, w2):
    gsched_jt, sched2, nact2 = pl.pallas_call(
        _router_kernel,
        out_shape=[
            jax.ShapeDtypeStruct((E, T), jnp.float32),
            jax.ShapeDtypeStruct((1, E), jnp.int32),
            jax.ShapeDtypeStruct((1, 1), jnp.int32),
        ],
    )(hidden_states, router_weight)
    sched = sched2[0]
    nact = nact2[0]
    gsched = gsched_jt.reshape(E, T, 1)

    def _wsel(i, s, n):
        return (s[jnp.minimum(i, n[0] - 1)], 0, 0)

    grid_spec = pltpu.PrefetchScalarGridSpec(
        num_scalar_prefetch=2,
        grid=(E,),
        in_specs=[
            pl.BlockSpec((T, D), lambda i, s, n: (0, 0)),
            pl.BlockSpec((1, 2 * F, D), _wsel),
            pl.BlockSpec((1, D, F), _wsel),
            pl.BlockSpec((1, T, 1), lambda i, s, n: (i, 0, 0)),
        ],
        out_specs=pl.BlockSpec((T, D), lambda i, s, n: (0, 0)),
    )
    return pl.pallas_call(
        _moe_kernel,
        grid_spec=grid_spec,
        out_shape=jax.ShapeDtypeStruct((T, D), jnp.float32),
        compiler_params=pltpu.CompilerParams(
            dimension_semantics=("arbitrary",)),
    )(sched, nact, hidden_states, w13, w2, gsched)


# speedup vs baseline: 2.7528x; 2.7528x over previous
import jax
import jax.numpy as jnp
from jax import lax
from jax.experimental import pallas as pl
from jax.experimental.pallas import tpu as pltpu

E = 64
TOPK = 2
D = 1024
F = 1024
T = 64

_HI = lax.Precision.HIGHEST


def _router_kernel(hid_ref, rw_ref, gsched_ref, sched_ref, nact_ref):
    hid = hid_ref[...]
    rw = rw_ref[...]
    logits = lax.dot_general(hid, rw, (((1,), (1,)), ((), ())),
                             preferred_element_type=jnp.float32,
                             precision=_HI)
    iota_e = lax.broadcasted_iota(jnp.int32, (T, E), 1)
    m1 = jnp.max(logits, axis=1, keepdims=True)
    i1 = jnp.min(jnp.where(logits == m1, iota_e, E), axis=1, keepdims=True)
    s1 = iota_e == i1
    masked = jnp.where(s1, -jnp.inf, logits)
    m2 = jnp.max(masked, axis=1, keepdims=True)
    i2 = jnp.min(jnp.where(masked == m2, iota_e, E), axis=1, keepdims=True)
    s2 = iota_e == i2
    e2 = jnp.exp(m2 - m1)
    w1 = 1.0 / (1.0 + e2)
    w2 = e2 / (1.0 + e2)
    gates = jnp.where(s1, w1, 0.0) + jnp.where(s2, w2, 0.0)

    sel = (s1 | s2).astype(jnp.float32)
    ones_t = jnp.ones((T, 1), jnp.float32)
    cnt_col = lax.dot_general(sel, ones_t, (((0,), (0,)), ((), ())),
                              precision=_HI)
    active_col = cnt_col > 0.5
    af = active_col.astype(jnp.float32)
    ii = lax.broadcasted_iota(jnp.int32, (E, E), 0)
    jj = lax.broadcasted_iota(jnp.int32, (E, E), 1)
    lt = (ii <= jj).astype(jnp.float32)
    incl_col = lax.dot_general(lt, af, (((0,), (0,)), ((), ())),
                               precision=_HI)
    excl_col = (incl_col - af).astype(jnp.int32)
    slot = lax.broadcasted_iota(jnp.int32, (E, E), 1)
    m_mat = ((excl_col == slot) & active_col).astype(jnp.float32)
    iota_col = lax.broadcasted_iota(jnp.int32, (E, 1), 0).astype(jnp.float32)
    sched_row = lax.dot_general(iota_col, m_mat, (((0,), (0,)), ((), ())),
                                precision=_HI)
    sched_ref[...] = sched_row.astype(jnp.int32)
    nact_ref[...] = incl_col[E - 1:E, :].astype(jnp.int32)
    gsched_ref[...] = lax.dot_general(m_mat, gates, (((0,), (1,)), ((), ())),
                                      precision=_HI)


def _moe_kernel(sched_sref, nact_sref, hid_ref, w13_ref, w2_ref, gsched_ref,
                out_ref):
    i = pl.program_id(0)

    @pl.when(i == 0)
    def _init():
        out_ref[...] = jnp.zeros_like(out_ref)

    @pl.when(i < nact_sref[0])
    def _compute():
        hid = hid_ref[...]
        h = lax.dot_general(hid, w13_ref[0], (((1,), (1,)), ((), ())),
                            preferred_element_type=jnp.float32)
        gatep = h[:, :F]
        up = h[:, F:]
        act = gatep * (1.0 / (1.0 + jnp.exp(-gatep))) * up
        y = lax.dot_general(act, w2_ref[0], (((1,), (1,)), ((), ())),
                            preferred_element_type=jnp.float32)
        g = gsched_ref[0]
        out_ref[...] += y * g


def kernel(hidden_states, router_weight, w13, w2):
    gsched_jt, sched2, nact2 = pl.pallas_call(
        _router_kernel,
        out_shape=[
            jax.ShapeDtypeStruct((E, T), jnp.float32),
            jax.ShapeDtypeStruct((1, E), jnp.int32),
            jax.ShapeDtypeStruct((1, 1), jnp.int32),
        ],
    )(hidden_states, router_weight)
    sched = sched2[0]
    nact = nact2[0]
    gsched = gsched_jt.reshape(E, T, 1)

    def _wsel(i, s, n):
        return (s[jnp.minimum(i, n[0] - 1)], 0, 0)

    grid_spec = pltpu.PrefetchScalarGridSpec(
        num_scalar_prefetch=2,
        grid=(E,),
        in_specs=[
            pl.BlockSpec((T, D), lambda i, s, n: (0, 0)),
            pl.BlockSpec((1, 2 * F, D), _wsel),
            pl.BlockSpec((1, D, F), _wsel),
            pl.BlockSpec((1, T, 1), lambda i, s, n: (i, 0, 0)),
        ],
        out_specs=pl.BlockSpec((T, D), lambda i, s, n: (0, 0)),
    )
    return pl.pallas_call(
        _moe_kernel,
        grid_spec=grid_spec,
        out_shape=jax.ShapeDtypeStruct((T, D), jnp.float32),
        compiler_params=pltpu.CompilerParams(
            dimension_semantics=("arbitrary",)),
    )(sched, nact, hidden_states, w13, w2, gsched)
